# Initial kernel scaffold; baseline (speedup 1.0000x reference)
import functools

import jax
import jax.numpy as jnp
from jax import lax
from jax.experimental import pallas as pl
from jax.experimental.pallas import tpu as pltpu
from jax.experimental.pallas import tpu_sc as plsc

NUM_CORES = 2
NUM_SUBCORES = 16
NUM_WORKERS = NUM_CORES * NUM_SUBCORES
CHUNK = 512


@functools.lru_cache(maxsize=None)
def _build(n_idx: int, dim: int):
    assert n_idx % (NUM_WORKERS * CHUNK) == 0
    per_worker = n_idx // NUM_WORKERS
    n_chunks = per_worker // CHUNK

    mesh = plsc.VectorSubcoreMesh(core_axis_name="c", subcore_axis_name="s")

    @functools.partial(
        pl.kernel,
        mesh=mesh,
        out_type=jax.ShapeDtypeStruct((n_idx, dim), jnp.float32),
        scratch_types=[
            pltpu.VMEM((CHUNK,), jnp.int32),
            pltpu.VMEM((CHUNK, dim), jnp.float32),
            pltpu.SemaphoreType.DMA,
        ],
    )
    def emb(table_hbm, idx_hbm, out_hbm, idx_v, rows_v, sem):
        wid = lax.axis_index("s") * NUM_CORES + lax.axis_index("c")
        base = wid * per_worker

        def chunk_body(c, carry):
            off = base + c * CHUNK
            pltpu.sync_copy(idx_hbm.at[pl.ds(off, CHUNK)], idx_v)
            pltpu.async_copy(table_hbm.at[idx_v], rows_v, sem).wait()
            pltpu.sync_copy(rows_v, out_hbm.at[pl.ds(off, CHUNK)])
            return carry

        lax.fori_loop(0, n_chunks, chunk_body, 0)

    return emb


def kernel(input, weight):
    b, h = input.shape
    dim = weight.shape[1]
    flat = input.reshape(-1).astype(jnp.int32)
    out = _build(flat.shape[0], dim)(weight, flat)
    return out.reshape(b, h, dim)


# speedup vs baseline: 1.0734x; 1.0734x over previous
import functools

import jax
import jax.numpy as jnp
from jax import lax
from jax.experimental import pallas as pl
from jax.experimental.pallas import tpu as pltpu
from jax.experimental.pallas import tpu_sc as plsc

NUM_CORES = 2
NUM_SUBCORES = 16
NUM_WORKERS = NUM_CORES * NUM_SUBCORES
CHUNK = 512


@functools.lru_cache(maxsize=None)
def _build(n_idx: int, dim: int):
    assert n_idx % (NUM_WORKERS * CHUNK) == 0
    per_worker = n_idx // NUM_WORKERS
    n_chunks = per_worker // CHUNK

    mesh = plsc.VectorSubcoreMesh(core_axis_name="c", subcore_axis_name="s")

    @functools.partial(
        pl.kernel,
        mesh=mesh,
        out_type=jax.ShapeDtypeStruct((n_idx, dim), jnp.float32),
        scratch_types=[
            pltpu.VMEM((CHUNK,), jnp.int32),
            pltpu.VMEM((CHUNK, dim), jnp.float32),
            pltpu.SemaphoreType.DMA,
        ],
        compiler_params=pltpu.CompilerParams(use_tc_tiling_on_sc=False),
    )
    def emb(table_hbm, idx_hbm, out_hbm, idx_v, rows_v, sem):
        wid = lax.axis_index("s") * NUM_CORES + lax.axis_index("c")
        base = wid * per_worker

        def chunk_body(c, carry):
            off = base + c * CHUNK
            pltpu.sync_copy(idx_hbm.at[pl.ds(off, CHUNK)], idx_v)
            pltpu.async_copy(table_hbm.at[idx_v], rows_v, sem).wait()
            pltpu.sync_copy(rows_v, out_hbm.at[pl.ds(off, CHUNK)])
            return carry

        lax.fori_loop(0, n_chunks, chunk_body, 0)

    return emb


def kernel(input, weight):
    b, h = input.shape
    dim = weight.shape[1]
    flat = input.reshape(-1).astype(jnp.int32)
    out = _build(flat.shape[0], dim)(weight, flat)
    return out.reshape(b, h, dim)


# speedup vs baseline: 1.1142x; 1.0380x over previous
import functools

import jax
import jax.numpy as jnp
from jax import lax
from jax.experimental import pallas as pl
from jax.experimental.pallas import tpu as pltpu
from jax.experimental.pallas import tpu_sc as plsc

NUM_CORES = 2
NUM_SUBCORES = 16
NUM_WORKERS = NUM_CORES * NUM_SUBCORES
CHUNK = 512
NBUF = 5


@functools.lru_cache(maxsize=None)
def _build(n_idx: int, dim: int):
    assert n_idx % (NUM_WORKERS * CHUNK * NBUF) == 0
    per_worker = n_idx // NUM_WORKERS
    n_chunks = per_worker // CHUNK
    n_outer = n_chunks // NBUF

    mesh = plsc.VectorSubcoreMesh(core_axis_name="c", subcore_axis_name="s")

    @functools.partial(
        pl.kernel,
        mesh=mesh,
        out_type=jax.ShapeDtypeStruct((n_idx, dim), jnp.float32),
        scratch_types=[
            pltpu.VMEM((per_worker,), jnp.int32),
            pltpu.VMEM((NBUF, CHUNK, dim), jnp.float32),
            [pltpu.SemaphoreType.DMA] * NBUF,
            [pltpu.SemaphoreType.DMA] * NBUF,
        ],
        compiler_params=pltpu.CompilerParams(use_tc_tiling_on_sc=False),
    )
    def emb(table_hbm, idx_hbm, out_hbm, idx_v, rows_v, gsems, wsems):
        wid = lax.axis_index("s") * NUM_CORES + lax.axis_index("c")
        base = wid * per_worker

        pltpu.sync_copy(idx_hbm.at[pl.ds(base, per_worker)], idx_v)

        def start_gather(c, b):
            pltpu.async_copy(
                table_hbm.at[idx_v.at[pl.ds(c * CHUNK, CHUNK)]],
                rows_v.at[b],
                gsems[b],
            )

        for b in range(NBUF):
            start_gather(b, b)

        def outer(o, carry):
            for b in range(NBUF):
                c = o * NBUF + b
                pltpu.make_async_copy(
                    table_hbm.at[idx_v.at[pl.ds(0, CHUNK)]], rows_v.at[b], gsems[b]
                ).wait()
                pltpu.async_copy(
                    rows_v.at[b],
                    out_hbm.at[pl.ds(base + c * CHUNK, CHUNK)],
                    wsems[b],
                )

                @pl.when(o < n_outer - 1)
                def _():
                    pltpu.make_async_copy(
                        rows_v.at[b], out_hbm.at[pl.ds(base, CHUNK)], wsems[b]
                    ).wait()
                    start_gather(c + NBUF, b)

            return carry

        lax.fori_loop(0, n_outer, outer, 0)

        for b in range(NBUF):
            pltpu.make_async_copy(
                rows_v.at[b], out_hbm.at[pl.ds(base, CHUNK)], wsems[b]
            ).wait()

    return emb


def kernel(input, weight):
    b, h = input.shape
    dim = weight.shape[1]
    flat = input.reshape(-1).astype(jnp.int32)
    out = _build(flat.shape[0], dim)(weight, flat)
    return out.reshape(b, h, dim)


# speedup vs baseline: 1.8127x; 1.6269x over previous
import functools

import jax
import jax.numpy as jnp
from jax import lax
from jax.experimental import pallas as pl
from jax.experimental.pallas import tpu as pltpu
from jax.experimental.pallas import tpu_sc as plsc

NUM_CORES = 2
NUM_SUBCORES = 16
NUM_WORKERS = NUM_CORES * NUM_SUBCORES
CHUNK = 512
NBUF = 5


@functools.lru_cache(maxsize=None)
def _build(n_idx: int, dim: int):
    assert n_idx % (NUM_WORKERS * CHUNK * NBUF) == 0
    per_worker = n_idx // NUM_WORKERS
    n_chunks = per_worker // CHUNK
    n_outer = n_chunks // NBUF

    mesh = plsc.VectorSubcoreMesh(core_axis_name="c", subcore_axis_name="s")

    @functools.partial(
        pl.kernel,
        mesh=mesh,
        out_type=jax.ShapeDtypeStruct((n_idx, dim), jnp.float32),
        scratch_types=[
            pltpu.VMEM((per_worker,), jnp.int32),
            pltpu.VMEM((NBUF, CHUNK, dim), jnp.float32),
            [pltpu.SemaphoreType.DMA] * NBUF,
            [pltpu.SemaphoreType.DMA] * NBUF,
        ],
        compiler_params=pltpu.CompilerParams(use_tc_tiling_on_sc=False),
    )
    def emb(table_hbm, idx_hbm, out_hbm, idx_v, rows_v, gsems, wsems):
        wid = lax.axis_index("s") * NUM_CORES + lax.axis_index("c")
        base = wid * per_worker

        pltpu.sync_copy(idx_hbm.at[pl.ds(base, per_worker)], idx_v)

        def start_gather(c, b):
            pltpu.async_copy(
                table_hbm.at[idx_v.at[pl.ds(c * CHUNK, CHUNK)]],
                rows_v.at[b],
                gsems[b],
            )

        for b in range(NBUF):
            start_gather(b, b)

        def outer(o, carry):
            for b in range(NBUF):
                c = o * NBUF + b
                pltpu.make_async_copy(
                    table_hbm.at[idx_v.at[pl.ds(0, CHUNK)]], rows_v.at[b], gsems[b]
                ).wait()
                pltpu.async_copy(
                    rows_v.at[b],
                    out_hbm.at[pl.ds(base + c * CHUNK, CHUNK)],
                    wsems[b],
                )

                @pl.when(o < n_outer - 1)
                def _():
                    pltpu.make_async_copy(
                        rows_v.at[b], out_hbm.at[pl.ds(base, CHUNK)], wsems[b]
                    ).wait()
                    start_gather(c + NBUF, b)

            return carry

        lax.fori_loop(0, n_outer, outer, 0)

        for b in range(NBUF):
            pltpu.make_async_copy(
                rows_v.at[b], out_hbm.at[pl.ds(base, CHUNK)], wsems[b]
            ).wait()

    return emb


def kernel(input, weight):
    b, h = input.shape
    n, dim = weight.shape
    flat = input.reshape(-1).astype(jnp.int32)
    w2 = lax.optimization_barrier(weight.reshape(-1)).reshape(n, dim)
    out = _build(flat.shape[0], dim)(w2, flat)
    return lax.optimization_barrier(out.reshape(-1)).reshape(b, h, dim)


# speedup vs baseline: 2.9469x; 1.6257x over previous
import functools

import jax
import jax.numpy as jnp
from jax import lax
from jax.experimental import pallas as pl
from jax.experimental.pallas import tpu as pltpu
from jax.experimental.pallas import tpu_sc as plsc

NUM_CORES = 2
NUM_SUBCORES = 16
NUM_WORKERS = NUM_CORES * NUM_SUBCORES
BLK = 128
SUB = 8


@functools.lru_cache(maxsize=None)
def _build(n_b: int, n_h: int, dim: int):
    assert n_b % (NUM_WORKERS * BLK) == 0 and dim % 16 == 0
    bpw = n_b // NUM_WORKERS
    nblk = bpw // BLK
    ntd = dim // SUB
    h_stride = dim * n_b
    td_stride = SUB * n_b

    mesh = plsc.VectorSubcoreMesh(core_axis_name="c", subcore_axis_name="s")

    @functools.partial(
        pl.kernel,
        mesh=mesh,
        out_type=jax.ShapeDtypeStruct((n_b * n_h * dim,), jnp.float32),
        scratch_types=[
            pltpu.VMEM((n_h, bpw), jnp.int32),
            pltpu.VMEM((nblk, BLK, dim), jnp.float32),
            pltpu.VMEM((nblk, dim * BLK), jnp.float32),
            [pltpu.SemaphoreType.DMA] * nblk,
            [pltpu.SemaphoreType.DMA] * nblk,
        ],
        compiler_params=pltpu.CompilerParams(use_tc_tiling_on_sc=False),
    )
    def emb(table_hbm, idxt_hbm, out_hbm, idx_v, a_v, t_v, gsems, wsems):
        wid = lax.axis_index("s") * NUM_CORES + lax.axis_index("c")
        b0 = wid * bpw

        pltpu.sync_copy(idxt_hbm.at[:, pl.ds(b0, bpw)], idx_v)

        lane = jax.lax.iota(jnp.int32, 16)
        stages = []
        for k in (1, 2, 4, 8):
            stages.append(((lane & k) != 0, lane ^ k, k))

        def start_gather(h, a):
            pltpu.async_copy(
                table_hbm.at[idx_v.at[h, pl.ds(a * BLK, BLK)]],
                a_v.at[a],
                gsems[a],
            )

        def wb_copy(h, a, td):
            return pltpu.make_async_copy(
                t_v.at[a].at[pl.ds(td * SUB * BLK, SUB * BLK)],
                out_hbm.at[
                    pl.ds(
                        h * h_stride + td * td_stride + (wid * nblk + a) * SUB * BLK,
                        SUB * BLK,
                    )
                ],
                wsems[a],
            )

        def transpose_block(a2, t1, bi, bj):
            r = [a2[bi * 16 + t, pl.ds(bj * 16, 16)] for t in range(16)]
            for mk, pm, k in stages:
                nr = list(r)
                for r0 in range(16):
                    if r0 & k:
                        continue
                    r1 = r0 ^ k
                    va, vb = r[r0], r[r1]
                    nr[r0] = jnp.where(mk, jnp.take(vb, pm), va)
                    nr[r1] = jnp.where(mk, vb, jnp.take(va, pm))
                r = nr
            for t in range(16):
                t1[pl.ds((bj * 16 + t) * BLK + bi * 16, 16)] = r[t]

        for a in range(nblk):
            start_gather(0, a)

        def outer(h, carry):
            for a in range(nblk):
                pltpu.make_async_copy(
                    table_hbm.at[idx_v.at[0, pl.ds(0, BLK)]], a_v.at[a], gsems[a]
                ).wait()

                @pl.when(h > 0)
                def _():
                    for td in range(ntd):
                        wb_copy(h, a, td).wait()

                a2 = a_v.at[a]
                t1 = t_v.at[a]

                def bi_body(bi, cc):
                    for bj in range(dim // 16):
                        transpose_block(a2, t1, bi, bj)
                    return cc

                lax.fori_loop(0, BLK // 16, bi_body, 0)

                @pl.when(h < n_h - 1)
                def _():
                    start_gather(h + 1, a)

                for td in range(ntd):
                    wb_copy(h, a, td).start()

            return carry

        lax.fori_loop(0, n_h, outer, 0)

        for a in range(nblk):
            for td in range(ntd):
                wb_copy(0, a, td).wait()

    return emb


def kernel(input, weight):
    b, h = input.shape
    n, dim = weight.shape
    idxt = lax.optimization_barrier(
        input.astype(jnp.int32).T.reshape(-1)
    ).reshape(h, b)
    w2 = lax.optimization_barrier(weight.reshape(-1)).reshape(n, dim)
    out = _build(b, h, dim)(w2, idxt)
    a = out.reshape(h, dim // SUB, b // BLK, SUB, BLK)
    return a.transpose(2, 4, 0, 1, 3).reshape(b, h, dim)


# speedup vs baseline: 5.5378x; 1.8792x over previous
import functools

import jax
import jax.numpy as jnp
from jax import lax
from jax.experimental import pallas as pl
from jax.experimental.pallas import tpu as pltpu
from jax.experimental.pallas import tpu_sc as plsc

NUM_CORES = 2
NUM_SUBCORES = 16
NUM_WORKERS = NUM_CORES * NUM_SUBCORES
BLK = 128
SUB = 8


def _stages():
    lane = jax.lax.iota(jnp.int32, 16)
    return [(((lane & k) != 0), lane ^ k, k) for k in (1, 2, 4, 8)]


def _butterfly16(r):
    for mk, pm, k in _stages():
        nr = list(r)
        for r0 in range(16):
            if r0 & k:
                continue
            r1 = r0 ^ k
            va, vb = r[r0], r[r1]
            nr[r0] = jnp.where(mk, jnp.take(vb, pm), va)
            nr[r1] = jnp.where(mk, vb, jnp.take(va, pm))
        r = nr
    return r


@functools.lru_cache(maxsize=None)
def _detile(n: int, dim: int):
    nfull = n // BLK
    tail = n - nfull * BLK
    niter = -(-nfull // NUM_WORKERS)
    nslots = -(-niter // 4) * 4

    mesh = plsc.VectorSubcoreMesh(core_axis_name="c", subcore_axis_name="s")

    @functools.partial(
        pl.kernel,
        mesh=mesh,
        out_type=jax.ShapeDtypeStruct((n * dim,), jnp.float32),
        scratch_types=[
            pltpu.VMEM((4, dim, BLK), jnp.float32),
            pltpu.VMEM((4, dim * BLK), jnp.float32),
            pltpu.VMEM((max(tail, 1) * dim,), jnp.float32),
            [pltpu.SemaphoreType.DMA] * 4,
            [pltpu.SemaphoreType.DMA] * 4,
        ],
    )
    def det(wt_hbm, wtail_hbm, wlin_hbm, ain, tout, tailv, isems, osems):
        wid = lax.axis_index("s") * NUM_CORES + lax.axis_index("c")

        if tail:
            @pl.when(wid == 0)
            def _():
                pltpu.sync_copy(wtail_hbm, tailv)
                pltpu.sync_copy(
                    tailv, wlin_hbm.at[pl.ds(nfull * BLK * dim, tail * dim)]
                )

        def jof(i):
            return wid + NUM_WORKERS * i

        def start_in(i, bA):
            pltpu.async_copy(
                wt_hbm.at[:, pl.ds(jof(i) * BLK, BLK)], ain.at[bA], isems[bA]
            )

        def out_copy(i, bA):
            return pltpu.make_async_copy(
                tout.at[bA],
                wlin_hbm.at[pl.ds(jof(i) * BLK * dim, BLK * dim)],
                osems[bA],
            )

        for bA in range(4):

            @pl.when(jof(bA) < nfull)
            def _():
                start_in(bA, bA)

        def outer(o, carry):
            for bA in range(4):
                i = o * 4 + bA
                j = jof(i)

                @pl.when(jnp.logical_and(o > 0, j - NUM_WORKERS * 4 < nfull))
                def _():
                    out_copy(i - 4, bA).wait()

                @pl.when(j < nfull)
                def _():
                    pltpu.make_async_copy(
                        wt_hbm.at[:, pl.ds(0, BLK)], ain.at[bA], isems[bA]
                    ).wait()
                    a2 = ain.at[bA]
                    t1 = tout.at[bA]

                    def br_body(br, cc):
                        for bd in range(dim // 16):
                            r = [
                                a2[bd * 16 + t, pl.ds(br * 16, 16)]
                                for t in range(16)
                            ]
                            r = _butterfly16(r)
                            for t in range(16):
                                t1[pl.ds((br * 16 + t) * dim + bd * 16, 16)] = r[t]
                        return cc

                    lax.fori_loop(0, BLK // 16, br_body, 0)

                    @pl.when(jof(i + 4) < nfull)
                    def _():
                        start_in(i + 4, bA)

                    out_copy(i, bA).start()

            return carry

        lax.fori_loop(0, nslots // 4, outer, 0)

        for bA in range(4):
            i = nslots - 4 + bA

            @pl.when(jof(i) < nfull)
            def _():
                out_copy(i, bA).wait()

    return det


@functools.lru_cache(maxsize=None)
def _build(n_b: int, n_h: int, dim: int):
    assert n_b % (NUM_WORKERS * BLK) == 0 and dim % 16 == 0
    bpw = n_b // NUM_WORKERS
    nblk = bpw // BLK
    ntd = dim // SUB
    h_stride = dim * n_b
    td_stride = SUB * n_b

    mesh = plsc.VectorSubcoreMesh(core_axis_name="c", subcore_axis_name="s")

    @functools.partial(
        pl.kernel,
        mesh=mesh,
        out_type=jax.ShapeDtypeStruct((n_b * n_h * dim,), jnp.float32),
        scratch_types=[
            pltpu.VMEM((n_h, bpw), jnp.int32),
            pltpu.VMEM((nblk, BLK, dim), jnp.float32),
            pltpu.VMEM((nblk, dim * BLK), jnp.float32),
            [pltpu.SemaphoreType.DMA] * nblk,
            [pltpu.SemaphoreType.DMA] * nblk,
        ],
        compiler_params=pltpu.CompilerParams(use_tc_tiling_on_sc=False),
    )
    def emb(table_hbm, idxt_hbm, out_hbm, idx_v, a_v, t_v, gsems, wsems):
        wid = lax.axis_index("s") * NUM_CORES + lax.axis_index("c")
        b0 = wid * bpw

        pltpu.sync_copy(idxt_hbm.at[:, pl.ds(b0, bpw)], idx_v)

        def start_gather(h, a):
            pltpu.async_copy(
                table_hbm.at[idx_v.at[h, pl.ds(a * BLK, BLK)]],
                a_v.at[a],
                gsems[a],
            )

        def wb_copy(h, a, td):
            return pltpu.make_async_copy(
                t_v.at[a].at[pl.ds(td * SUB * BLK, SUB * BLK)],
                out_hbm.at[
                    pl.ds(
                        h * h_stride + td * td_stride + (wid * nblk + a) * SUB * BLK,
                        SUB * BLK,
                    )
                ],
                wsems[a],
            )

        for a in range(nblk):
            start_gather(0, a)

        def outer(h, carry):
            for a in range(nblk):
                pltpu.make_async_copy(
                    table_hbm.at[idx_v.at[0, pl.ds(0, BLK)]], a_v.at[a], gsems[a]
                ).wait()

                @pl.when(h > 0)
                def _():
                    for td in range(ntd):
                        wb_copy(h, a, td).wait()

                a2 = a_v.at[a]
                t1 = t_v.at[a]

                def bi_body(bi, cc):
                    for bj in range(dim // 16):
                        r = [
                            a2[bi * 16 + t, pl.ds(bj * 16, 16)]
                            for t in range(16)
                        ]
                        r = _butterfly16(r)
                        for t in range(16):
                            t1[pl.ds((bj * 16 + t) * BLK + bi * 16, 16)] = r[t]
                    return cc

                lax.fori_loop(0, BLK // 16, bi_body, 0)

                @pl.when(h < n_h - 1)
                def _():
                    start_gather(h + 1, a)

                for td in range(ntd):
                    wb_copy(h, a, td).start()

            return carry

        lax.fori_loop(0, n_h, outer, 0)

        for a in range(nblk):
            for td in range(ntd):
                wb_copy(0, a, td).wait()

    return emb


def kernel(input, weight):
    b, h = input.shape
    n, dim = weight.shape
    idxt = lax.optimization_barrier(
        input.astype(jnp.int32).T.reshape(-1)
    ).reshape(h, b)
    nfull = (n // BLK) * BLK
    wtail = lax.optimization_barrier(weight[nfull:].reshape(-1))
    wlin = _detile(n, dim)(weight.T, wtail)
    w2 = wlin.reshape(n, dim)
    out = _build(b, h, dim)(w2, idxt)
    a = out.reshape(h, dim // SUB, b // BLK, SUB, BLK)
    return a.transpose(2, 4, 0, 1, 3).reshape(b, h, dim)
